# Initial kernel scaffold; baseline (speedup 1.0000x reference)
import jax
import jax.numpy as jnp
from jax.experimental import pallas as pl
from jax.experimental.pallas import tpu as pltpu
from jax.experimental.pallas import tpu_sc as plsc

_EMB = 32
_WINDOW = 128


def kernel(inputs, table):
    n_rows, n_cols = inputs.shape
    total = n_rows * n_cols
    flat_idx = inputs.reshape(1, total).astype(jnp.int32)
    mesh = plsc.VectorSubcoreMesh(
        core_axis_name="core", subcore_axis_name="subcore"
    )

    @pl.kernel(
        out_type=jax.ShapeDtypeStruct((total, _EMB), table.dtype),
        mesh=mesh,
    )
    def _gather(table_hbm, idx_hbm, out_hbm):
        def body(idx_vmem, out_vmem):
            pltpu.sync_copy(table_hbm.at[idx_vmem.at[0]], out_vmem)

        pltpu.emit_pipeline(
            body,
            grid=(total // _WINDOW,),
            in_specs=[pl.BlockSpec((1, _WINDOW), index_map=lambda i: (0, i))],
            out_specs=[
                pl.BlockSpec((_WINDOW, _EMB), index_map=lambda i: (i, 0))
            ],
            core_axis_name=("core", "subcore"),
            dimension_semantics=(pltpu.PARALLEL,),
        )(idx_hbm, out_hbm)

    out = _gather(table, flat_idx)
    return out.reshape(n_rows, n_cols, _EMB)


# speedup vs baseline: 1.0432x; 1.0432x over previous
import jax
import jax.numpy as jnp
from jax.experimental import pallas as pl
from jax.experimental.pallas import tpu as pltpu
from jax.experimental.pallas import tpu_sc as plsc

_EMB = 32
_WINDOW = 128


def kernel(inputs, table):
    n_rows, n_cols = inputs.shape
    total = n_rows * n_cols
    flat_idx = inputs.reshape(1, total).astype(jnp.int32)
    mesh = plsc.VectorSubcoreMesh(
        core_axis_name="core", subcore_axis_name="subcore"
    )

    @pl.kernel(
        out_type=jax.ShapeDtypeStruct((total, _EMB), table.dtype),
        mesh=mesh,
        compiler_params=pltpu.CompilerParams(use_tc_tiling_on_sc=False),
    )
    def _gather(table_hbm, idx_hbm, out_hbm):
        def body(idx_vmem, out_vmem):
            pltpu.sync_copy(table_hbm.at[idx_vmem.at[0]], out_vmem)

        pltpu.emit_pipeline(
            body,
            grid=(total // _WINDOW,),
            in_specs=[pl.BlockSpec((1, _WINDOW), index_map=lambda i: (0, i))],
            out_specs=[
                pl.BlockSpec((_WINDOW, _EMB), index_map=lambda i: (i, 0))
            ],
            core_axis_name=("core", "subcore"),
            dimension_semantics=(pltpu.PARALLEL,),
        )(idx_hbm, out_hbm)

    out = _gather(table, flat_idx)
    return out.reshape(n_rows, n_cols, _EMB)


# speedup vs baseline: 1.1001x; 1.0545x over previous
import jax
import jax.numpy as jnp
from jax.experimental import pallas as pl
from jax.experimental.pallas import tpu as pltpu
from jax.experimental.pallas import tpu_sc as plsc

_EMB = 32
_WINDOW = 512


def kernel(inputs, table):
    n_rows, n_cols = inputs.shape
    total = n_rows * n_cols
    flat_idx = inputs.reshape(1, total).astype(jnp.int32)
    mesh = plsc.VectorSubcoreMesh(
        core_axis_name="core", subcore_axis_name="subcore"
    )

    @pl.kernel(
        out_type=jax.ShapeDtypeStruct((total, _EMB), table.dtype),
        mesh=mesh,
        compiler_params=pltpu.CompilerParams(use_tc_tiling_on_sc=False),
    )
    def _gather(table_hbm, idx_hbm, out_hbm):
        def body(idx_vmem, out_vmem):
            pltpu.sync_copy(table_hbm.at[idx_vmem.at[0]], out_vmem)

        pltpu.emit_pipeline(
            body,
            grid=(total // _WINDOW,),
            in_specs=[pl.BlockSpec((1, _WINDOW), index_map=lambda i: (0, i))],
            out_specs=[
                pl.BlockSpec((_WINDOW, _EMB), index_map=lambda i: (i, 0))
            ],
            core_axis_name=("core", "subcore"),
            dimension_semantics=(pltpu.PARALLEL,),
        )(idx_hbm, out_hbm)

    out = _gather(table, flat_idx)
    return out.reshape(n_rows, n_cols, _EMB)


# speedup vs baseline: 1.4289x; 1.2988x over previous
import jax
import jax.numpy as jnp
from jax.experimental import pallas as pl
from jax.experimental.pallas import tpu as pltpu
from jax.experimental.pallas import tpu_sc as plsc

_EMB = 32
_ROWS_PER_BLOCK = 8


def kernel(inputs, table):
    n_rows, n_cols = inputs.shape
    mesh = plsc.VectorSubcoreMesh(
        core_axis_name="core", subcore_axis_name="subcore"
    )
    R = _ROWS_PER_BLOCK

    @pl.kernel(
        out_type=jax.ShapeDtypeStruct((n_rows, n_cols, _EMB), table.dtype),
        mesh=mesh,
        compiler_params=pltpu.CompilerParams(use_tc_tiling_on_sc=False),
    )
    def _gather(table_hbm, idx_hbm, out_hbm):
        def body(idx_vmem, out_vmem):
            for j in range(R):
                pltpu.sync_copy(
                    table_hbm.at[idx_vmem.at[j]], out_vmem.at[j]
                )

        pltpu.emit_pipeline(
            body,
            grid=(n_rows // R,),
            in_specs=[pl.BlockSpec((R, n_cols), index_map=lambda i: (i, 0))],
            out_specs=[
                pl.BlockSpec(
                    (R, n_cols, _EMB), index_map=lambda i: (i, 0, 0)
                )
            ],
            core_axis_name=("core", "subcore"),
            dimension_semantics=(pltpu.PARALLEL,),
        )(idx_hbm, out_hbm)

    return _gather(table, inputs)


# speedup vs baseline: 1.7412x; 1.2186x over previous
import jax
import jax.numpy as jnp
from jax import lax
from jax.experimental import pallas as pl
from jax.experimental.pallas import tpu as pltpu
from jax.experimental.pallas import tpu_sc as plsc

_EMB = 32
_NUM_CORES = 2
_NUM_SUBCORES = 16
_R = 16


def kernel(inputs, table):
    n_rows, n_cols = inputs.shape
    n_workers = _NUM_CORES * _NUM_SUBCORES
    rows_per_worker = n_rows // n_workers
    steps = rows_per_worker // _R

    mesh = plsc.VectorSubcoreMesh(
        core_axis_name="core", subcore_axis_name="subcore"
    )

    @pl.kernel(
        out_type=jax.ShapeDtypeStruct((n_rows, n_cols, _EMB), table.dtype),
        mesh=mesh,
        scratch_types=[
            pltpu.VMEM((_R, n_cols), jnp.int32),
            pltpu.VMEM((_R, n_cols, _EMB), jnp.float32),
            pltpu.SemaphoreType.DMA,
        ],
        compiler_params=pltpu.CompilerParams(use_tc_tiling_on_sc=False),
    )
    def _gather(table_hbm, idx_hbm, out_hbm, idx_v, rows_v, sem):
        wid = lax.axis_index("subcore") * _NUM_CORES + lax.axis_index("core")
        row_base = wid * rows_per_worker

        @pl.loop(0, steps)
        def _(i):
            r0 = row_base + i * _R
            pltpu.sync_copy(idx_hbm.at[pl.ds(r0, _R)], idx_v)
            copies = [
                pltpu.async_copy(
                    table_hbm.at[idx_v.at[j]], rows_v.at[j], sem
                )
                for j in range(_R)
            ]
            for c in copies:
                c.wait()
            pltpu.sync_copy(rows_v, out_hbm.at[pl.ds(r0, _R)])

    return _gather(table, inputs)
